# Initial kernel scaffold; baseline (speedup 1.0000x reference)
import jax
import jax.numpy as jnp
from jax.experimental import pallas as pl


def kernel(x, edge_index, W_gcn, b_gcn, W1, b1, W2, b2, g1, be1, g2, be2):
    raise NotImplementedError("write your pallas kernel here")


# speedup vs baseline: 15.8890x; 15.8890x over previous
import functools

import jax
import jax.numpy as jnp
from jax import lax
from jax.experimental import pallas as pl
from jax.experimental.pallas import tpu as pltpu
from jax.experimental.pallas import tpu_sc as plsc

N = 10000
E = 320000
D = 128
H = 128
FF = 256

NC = 2
NS = 16
CHUNK = 80
EDGES_PER_TILE = E // (NC * NS)
NCHUNKS = EDGES_PER_TILE // CHUNK
NPAD = 10240
ROWS_PER_TILE = NPAD // NS
DEG_PAD = 10240
DEG_TILE = DEG_PAD // NS

ROW_BLOCK = 1000
GRID = N // ROW_BLOCK

_MESH = plsc.VectorSubcoreMesh(
    core_axis_name="c", subcore_axis_name="s", num_cores=NC, num_subcores=NS)


@functools.partial(
    pl.kernel,
    mesh=_MESH,
    out_type=jax.ShapeDtypeStruct((NC, DEG_PAD), jnp.float32),
    scratch_types=[
        pltpu.VMEM((CHUNK,), jnp.int32),
        pltpu.VMEM((CHUNK,), jnp.float32),
        pltpu.VMEM_SHARED((DEG_PAD,), jnp.float32),
    ],
)
def _sc_degree(dst_hbm, z_hbm, out_hbm, idx_v, ones_v, acc_sh):
    cid = lax.axis_index("c")
    sid = lax.axis_index("s")

    def ones_body(i, c):
        ones_v[pl.ds(i * 16, 16)] = jnp.ones((16,), jnp.float32)
        return c
    lax.fori_loop(0, CHUNK // 16, ones_body, 0)

    pltpu.sync_copy(z_hbm, acc_sh.at[pl.ds(sid * DEG_TILE, DEG_TILE)])
    plsc.subcore_barrier()

    base = (cid * NS + sid) * EDGES_PER_TILE

    def body(i, c):
        pltpu.sync_copy(dst_hbm.at[pl.ds(base + i * CHUNK, CHUNK)], idx_v)
        pltpu.sync_copy(ones_v, acc_sh.at[idx_v], add=True)
        return c
    lax.fori_loop(0, NCHUNKS, body, 0)

    plsc.subcore_barrier()
    pltpu.sync_copy(acc_sh.at[pl.ds(sid * DEG_TILE, DEG_TILE)],
                    out_hbm.at[cid, pl.ds(sid * DEG_TILE, DEG_TILE)])


@functools.partial(
    pl.kernel,
    mesh=_MESH,
    out_type=jax.ShapeDtypeStruct((NC, NPAD, H), jnp.float32),
    scratch_types=[
        pltpu.VMEM((CHUNK,), jnp.int32),
        pltpu.VMEM((CHUNK,), jnp.int32),
        pltpu.VMEM((CHUNK, H), jnp.float32),
        pltpu.VMEM_SHARED((NPAD, H), jnp.float32),
    ],
)
def _sc_scatter(hs_hbm, src_hbm, dst_hbm, z_hbm, out_hbm, srcv, dstv, rows_v, acc_sh):
    cid = lax.axis_index("c")
    sid = lax.axis_index("s")

    pltpu.sync_copy(z_hbm, acc_sh.at[pl.ds(sid * ROWS_PER_TILE, ROWS_PER_TILE)])
    plsc.subcore_barrier()

    base = (cid * NS + sid) * EDGES_PER_TILE

    def body(i, c):
        off = base + i * CHUNK
        pltpu.sync_copy(src_hbm.at[pl.ds(off, CHUNK)], srcv)
        pltpu.sync_copy(dst_hbm.at[pl.ds(off, CHUNK)], dstv)
        pltpu.sync_copy(hs_hbm.at[srcv], rows_v)
        pltpu.sync_copy(rows_v, acc_sh.at[dstv], add=True)
        return c
    lax.fori_loop(0, NCHUNKS, body, 0)

    plsc.subcore_barrier()
    pltpu.sync_copy(acc_sh.at[pl.ds(sid * ROWS_PER_TILE, ROWS_PER_TILE)],
                    out_hbm.at[cid, pl.ds(sid * ROWS_PER_TILE, ROWS_PER_TILE)])


def _tc1_body(x_ref, w_ref, deg_ref, hs_ref):
    h = jnp.dot(x_ref[...], w_ref[...], preferred_element_type=jnp.float32)
    dinv = lax.rsqrt(deg_ref[...])
    hs_ref[...] = h * dinv


def _tc_scale(x, W_gcn, deg2d):
    return pl.pallas_call(
        _tc1_body,
        grid=(GRID,),
        in_specs=[
            pl.BlockSpec((ROW_BLOCK, D), lambda i: (i, 0)),
            pl.BlockSpec((D, H), lambda i: (0, 0)),
            pl.BlockSpec((ROW_BLOCK, 1), lambda i: (i, 0)),
        ],
        out_specs=pl.BlockSpec((ROW_BLOCK, H), lambda i: (i, 0)),
        out_shape=jax.ShapeDtypeStruct((N, H), jnp.float32),
    )(x, W_gcn, deg2d)


def _ln(v, gamma, beta, eps=1e-5):
    mu = jnp.mean(v, axis=-1, keepdims=True)
    var = jnp.mean((v - mu) * (v - mu), axis=-1, keepdims=True)
    return (v - mu) * lax.rsqrt(var + eps) * gamma + beta


def _tc2_body(sp_ref, hs_ref, deg_ref, x_ref, bg_ref, w1_ref, b1_ref, w2_ref,
              b2_ref, g1_ref, be1_ref, g2_ref, be2_ref, out_ref):
    s = sp_ref[0] + sp_ref[1]
    dinv = lax.rsqrt(deg_ref[...])
    agg = dinv * (s + hs_ref[...]) + bg_ref[...]
    xr = x_ref[...] + agg
    xn = _ln(xr, g1_ref[...], be1_ref[...])
    t = jnp.maximum(
        jnp.dot(xn, w1_ref[...], preferred_element_type=jnp.float32) + b1_ref[...],
        0.0)
    ff = jnp.dot(t, w2_ref[...], preferred_element_type=jnp.float32) + b2_ref[...]
    out_ref[...] = _ln(xn + ff, g2_ref[...], be2_ref[...])


def _tc_epilogue(sp, hs, deg2d, x, b_gcn, W1, b1, W2, b2, g1, be1, g2, be2):
    full = lambda shape: pl.BlockSpec(shape, lambda i: tuple(0 for _ in shape))
    return pl.pallas_call(
        _tc2_body,
        grid=(GRID,),
        in_specs=[
            pl.BlockSpec((NC, ROW_BLOCK, H), lambda i: (0, i, 0)),
            pl.BlockSpec((ROW_BLOCK, H), lambda i: (i, 0)),
            pl.BlockSpec((ROW_BLOCK, 1), lambda i: (i, 0)),
            pl.BlockSpec((ROW_BLOCK, D), lambda i: (i, 0)),
            full((H,)),
            full((H, FF)),
            full((FF,)),
            full((FF, H)),
            full((H,)),
            full((H,)),
            full((H,)),
            full((H,)),
            full((H,)),
        ],
        out_specs=pl.BlockSpec((ROW_BLOCK, H), lambda i: (i, 0)),
        out_shape=jax.ShapeDtypeStruct((N, H), jnp.float32),
    )(sp, hs, deg2d, x, b_gcn, W1, b1, W2, b2, g1, be1, g2, be2)


def kernel(x, edge_index, W_gcn, b_gcn, W1, b1, W2, b2, g1, be1, g2, be2):
    src = edge_index[0].astype(jnp.int32)
    dst = edge_index[1].astype(jnp.int32)

    degp = _sc_degree(dst, jnp.zeros((DEG_TILE,), jnp.float32))
    deg2d = (degp[0, :N] + degp[1, :N] + 1.0).reshape(N, 1)

    hs = _tc_scale(x, W_gcn, deg2d)

    sp = _sc_scatter(hs, src, dst,
                     jnp.zeros((ROWS_PER_TILE, H), jnp.float32))

    return _tc_epilogue(sp, hs, deg2d, x, b_gcn, W1, b1, W2, b2,
                        g1, be1, g2, be2)


# speedup vs baseline: 29.9722x; 1.8863x over previous
import functools

import jax
import jax.numpy as jnp
from jax import lax
from jax.experimental import pallas as pl
from jax.experimental.pallas import tpu as pltpu
from jax.experimental.pallas import tpu_sc as plsc

N = 10000
E = 320000
D = 128
H = 128
FF = 256

NC = 2
NS = 16
CHUNK = 80
EDGES_PER_TILE = E // (NC * NS)
NCHUNKS = EDGES_PER_TILE // CHUNK
NPAD = 10240
ROWS_PER_TILE = NPAD // NS
DEG_PAD = 10240
DEG_TILE = DEG_PAD // NS

ROW_BLOCK = 1000
GRID = N // ROW_BLOCK

_MESH = plsc.VectorSubcoreMesh(
    core_axis_name="c", subcore_axis_name="s", num_cores=NC, num_subcores=NS)


@functools.partial(
    pl.kernel,
    mesh=_MESH,
    out_type=jax.ShapeDtypeStruct((NC, DEG_PAD), jnp.float32),
    scratch_types=[
        pltpu.VMEM((NCHUNKS, CHUNK), jnp.int32),
        pltpu.VMEM((CHUNK,), jnp.float32),
        pltpu.SemaphoreType.DMA,
        pltpu.VMEM_SHARED((DEG_PAD,), jnp.float32),
    ],
)
def _sc_degree(dst_hbm, z_hbm, out_hbm, dstv, ones_v, ssem, acc_sh):
    cid = lax.axis_index("c")
    sid = lax.axis_index("s")
    wid = cid * NS + sid

    def ones_body(i, c):
        ones_v[pl.ds(i * 16, 16)] = jnp.ones((16,), jnp.float32)
        return c
    lax.fori_loop(0, CHUNK // 16, ones_body, 0)

    pltpu.sync_copy(dst_hbm.at[wid], dstv)
    pltpu.sync_copy(z_hbm, acc_sh.at[pl.ds(sid * DEG_TILE, DEG_TILE)])
    plsc.subcore_barrier()

    def body(i, c):
        pltpu.async_copy(ones_v, acc_sh.at[dstv.at[i]], ssem, add=True)
        return c
    lax.fori_loop(0, NCHUNKS, body, 0)

    def drain(i, c):
        pltpu.make_async_copy(ones_v, acc_sh.at[dstv.at[i]], ssem).wait()
        return c
    lax.fori_loop(0, NCHUNKS, drain, 0)

    plsc.subcore_barrier()
    pltpu.sync_copy(acc_sh.at[pl.ds(sid * DEG_TILE, DEG_TILE)],
                    out_hbm.at[cid, pl.ds(sid * DEG_TILE, DEG_TILE)])


@functools.partial(
    pl.kernel,
    mesh=_MESH,
    out_type=jax.ShapeDtypeStruct((NC, NPAD, H), jnp.float32),
    scratch_types=[
        pltpu.VMEM((3, CHUNK), jnp.int32),
        pltpu.VMEM((NCHUNKS, CHUNK), jnp.int32),
        pltpu.VMEM((2, CHUNK, H), jnp.float32),
        pltpu.SemaphoreType.DMA,
        pltpu.SemaphoreType.DMA,
        pltpu.VMEM_SHARED((NPAD, H), jnp.float32),
    ],
)
def _sc_scatter(hs_hbm, src_hbm, dst_hbm, z_hbm, out_hbm, srcv, dstv, rows_v,
                isem, gsem, acc_sh):
    cid = lax.axis_index("c")
    sid = lax.axis_index("s")
    wid = cid * NS + sid

    pltpu.sync_copy(dst_hbm.at[wid], dstv)
    pltpu.sync_copy(z_hbm, acc_sh.at[pl.ds(sid * ROWS_PER_TILE, ROWS_PER_TILE)])
    plsc.subcore_barrier()

    pltpu.sync_copy(src_hbm.at[wid, 0], srcv.at[0])
    pltpu.async_copy(hs_hbm.at[srcv.at[0]], rows_v.at[0], gsem)
    pltpu.async_copy(src_hbm.at[wid, 1], srcv.at[1], isem)

    def body(i, c):
        b = lax.rem(i, 2)
        pltpu.make_async_copy(hs_hbm.at[srcv.at[lax.rem(i, 3)]],
                              rows_v.at[b], gsem).wait()

        @pl.when(i + 1 < NCHUNKS)
        def _():
            m = lax.rem(i + 1, 3)
            pltpu.make_async_copy(src_hbm.at[wid, i + 1], srcv.at[m], isem).wait()
            pltpu.async_copy(hs_hbm.at[srcv.at[m]], rows_v.at[1 - b], gsem)

        @pl.when(i + 2 < NCHUNKS)
        def _():
            pltpu.async_copy(src_hbm.at[wid, i + 2],
                             srcv.at[lax.rem(i + 2, 3)], isem)

        pltpu.sync_copy(rows_v.at[b], acc_sh.at[dstv.at[i]], add=True)
        return c
    lax.fori_loop(0, NCHUNKS, body, 0)

    plsc.subcore_barrier()
    pltpu.sync_copy(acc_sh.at[pl.ds(sid * ROWS_PER_TILE, ROWS_PER_TILE)],
                    out_hbm.at[cid, pl.ds(sid * ROWS_PER_TILE, ROWS_PER_TILE)])


def _tc1_body(x_ref, w_ref, deg_ref, hs_ref):
    h = jnp.dot(x_ref[...], w_ref[...], preferred_element_type=jnp.float32)
    dinv = lax.rsqrt(deg_ref[...])
    hs_ref[...] = h * dinv


def _tc_scale(x, W_gcn, deg2d):
    return pl.pallas_call(
        _tc1_body,
        grid=(GRID,),
        in_specs=[
            pl.BlockSpec((ROW_BLOCK, D), lambda i: (i, 0)),
            pl.BlockSpec((D, H), lambda i: (0, 0)),
            pl.BlockSpec((ROW_BLOCK, 1), lambda i: (i, 0)),
        ],
        out_specs=pl.BlockSpec((ROW_BLOCK, H), lambda i: (i, 0)),
        out_shape=jax.ShapeDtypeStruct((N, H), jnp.float32),
    )(x, W_gcn, deg2d)


def _ln(v, gamma, beta, eps=1e-5):
    mu = jnp.mean(v, axis=-1, keepdims=True)
    var = jnp.mean((v - mu) * (v - mu), axis=-1, keepdims=True)
    return (v - mu) * lax.rsqrt(var + eps) * gamma + beta


def _tc2_body(sp_ref, hs_ref, deg_ref, x_ref, bg_ref, w1_ref, b1_ref, w2_ref,
              b2_ref, g1_ref, be1_ref, g2_ref, be2_ref, out_ref):
    s = sp_ref[0] + sp_ref[1]
    dinv = lax.rsqrt(deg_ref[...])
    agg = dinv * (s + hs_ref[...]) + bg_ref[...]
    xr = x_ref[...] + agg
    xn = _ln(xr, g1_ref[...], be1_ref[...])
    t = jnp.maximum(
        jnp.dot(xn, w1_ref[...], preferred_element_type=jnp.float32) + b1_ref[...],
        0.0)
    ff = jnp.dot(t, w2_ref[...], preferred_element_type=jnp.float32) + b2_ref[...]
    out_ref[...] = _ln(xn + ff, g2_ref[...], be2_ref[...])


def _tc_epilogue(sp, hs, deg2d, x, b_gcn, W1, b1, W2, b2, g1, be1, g2, be2):
    full = lambda shape: pl.BlockSpec(shape, lambda i: tuple(0 for _ in shape))
    return pl.pallas_call(
        _tc2_body,
        grid=(GRID,),
        in_specs=[
            pl.BlockSpec((NC, ROW_BLOCK, H), lambda i: (0, i, 0)),
            pl.BlockSpec((ROW_BLOCK, H), lambda i: (i, 0)),
            pl.BlockSpec((ROW_BLOCK, 1), lambda i: (i, 0)),
            pl.BlockSpec((ROW_BLOCK, D), lambda i: (i, 0)),
            full((H,)),
            full((H, FF)),
            full((FF,)),
            full((FF, H)),
            full((H,)),
            full((H,)),
            full((H,)),
            full((H,)),
            full((H,)),
        ],
        out_specs=pl.BlockSpec((ROW_BLOCK, H), lambda i: (i, 0)),
        out_shape=jax.ShapeDtypeStruct((N, H), jnp.float32),
    )(sp, hs, deg2d, x, b_gcn, W1, b1, W2, b2, g1, be1, g2, be2)


def kernel(x, edge_index, W_gcn, b_gcn, W1, b1, W2, b2, g1, be1, g2, be2):
    src3 = edge_index[0].astype(jnp.int32).reshape(NC * NS, NCHUNKS, CHUNK)
    dst3 = edge_index[1].astype(jnp.int32).reshape(NC * NS, NCHUNKS, CHUNK)

    degp = _sc_degree(dst3, jnp.zeros((DEG_TILE,), jnp.float32))
    deg2d = (degp[0, :N] + degp[1, :N] + 1.0).reshape(N, 1)

    hs = _tc_scale(x, W_gcn, deg2d)

    sp = _sc_scatter(hs, src3, dst3,
                     jnp.zeros((ROWS_PER_TILE, H), jnp.float32))

    return _tc_epilogue(sp, hs, deg2d, x, b_gcn, W1, b1, W2, b2,
                        g1, be1, g2, be2)
